# initial kernel scaffold (unmeasured)
import jax
import jax.numpy as jnp
from jax import lax
from jax.experimental import pallas as pl
from jax.experimental.pallas import tpu as pltpu

N_DEV = 4
B, SQ, DM = 2, 128, 512
HL, DH = 4, 64
SKV_SH = 128
WIN = 128
BF16 = jnp.bfloat16


def kernel(x, Wq, K_ext, V_ext, Wo):
    xb = x.astype(BF16)
    wqb = Wq.astype(BF16)
    wob = Wo.astype(BF16)
    kb = jnp.transpose(K_ext.astype(BF16), (2, 0, 1, 3))
    vb = jnp.transpose(V_ext.astype(BF16), (2, 0, 1, 3))

    def body(x_ref, wq_ref, k_ref, v_ref, wo_ref, out_ref,
             kv_comm, out_comm, kv_send, kv_recv, o_send, o_recv):
        my = lax.axis_index("i")
        left = lax.rem(my + N_DEV - 1, N_DEV)
        right = lax.rem(my + 1, N_DEV)

        barrier = pltpu.get_barrier_semaphore()
        for nbr in (left, right):
            pl.semaphore_signal(barrier, inc=1, device_id=(nbr,),
                                device_id_type=pl.DeviceIdType.MESH)
        pl.semaphore_wait(barrier, 2)

        kv_comm[0, 0] = k_ref[...]
        kv_comm[0, 1] = v_ref[...]

        qh = None
        for h in range(N_DEV - 1):
            rdma = pltpu.make_async_remote_copy(
                src_ref=kv_comm.at[h],
                dst_ref=kv_comm.at[h + 1],
                send_sem=kv_send.at[h],
                recv_sem=kv_recv.at[h],
                device_id=(right,),
                device_id_type=pl.DeviceIdType.MESH,
            )
            rdma.start()
            if h == 0:
                q = jnp.dot(x_ref[...].reshape(B * SQ, DM), wq_ref[...],
                            preferred_element_type=jnp.float32)
                qh = q.astype(BF16).reshape(B, SQ, HL, DH).transpose(2, 0, 1, 3)
            rdma.wait()

        h0 = HL * my
        s0 = my
        s1 = left
        k0 = kv_comm[pl.ds(s0, 1), 0, pl.ds(h0, HL)].reshape(HL, B, SKV_SH, DH)
        k1 = kv_comm[pl.ds(s1, 1), 0, pl.ds(h0, HL)].reshape(HL, B, SKV_SH, DH)
        v0 = kv_comm[pl.ds(s0, 1), 1, pl.ds(h0, HL)].reshape(HL, B, SKV_SH, DH)
        v1 = kv_comm[pl.ds(s1, 1), 1, pl.ds(h0, HL)].reshape(HL, B, SKV_SH, DH)
        k01 = jnp.concatenate([k0, k1], axis=2)
        v01 = jnp.concatenate([v0, v1], axis=2)

        scores = jnp.einsum('hbsd,hbtd->hbst', qh, k01,
                            preferred_element_type=jnp.float32) * 0.125
        si = lax.broadcasted_iota(jnp.int32, (SQ, 2 * SKV_SH), 0)
        ti = lax.broadcasted_iota(jnp.int32, (SQ, 2 * SKV_SH), 1)
        mask = (ti - si) <= WIN
        scores = jnp.where(mask[None, None], scores, -1e9)

        m = jnp.max(scores, axis=-1, keepdims=True)
        e = jnp.exp(scores - m)
        w = (e / jnp.sum(e, axis=-1, keepdims=True)).astype(BF16)

        ctx = jnp.einsum('hbst,hbtd->hbsd', w, v01,
                         preferred_element_type=jnp.float32)
        ctx2 = ctx.astype(BF16).transpose(1, 2, 0, 3).reshape(B * SQ, HL * DH)
        part = jnp.dot(ctx2, wo_ref[...], preferred_element_type=jnp.float32)
        out_comm[0] = part.reshape(B, SQ, DM)

        for h in range(N_DEV - 1):
            rdma = pltpu.make_async_remote_copy(
                src_ref=out_comm.at[h],
                dst_ref=out_comm.at[h + 1],
                send_sem=o_send.at[h],
                recv_sem=o_recv.at[h],
                device_id=(right,),
                device_id_type=pl.DeviceIdType.MESH,
            )
            rdma.start()
            rdma.wait()

        out_ref[...] = ((out_comm[0] + out_comm[1])
                        + (out_comm[2] + out_comm[3]))

    return pl.pallas_call(
        body,
        out_shape=jax.ShapeDtypeStruct((B, SQ, DM), jnp.float32),
        in_specs=[pl.BlockSpec(memory_space=pltpu.VMEM)] * 5,
        out_specs=pl.BlockSpec(memory_space=pltpu.VMEM),
        scratch_shapes=[
            pltpu.VMEM((N_DEV, 2, N_DEV * HL, B, SKV_SH, DH), BF16),
            pltpu.VMEM((N_DEV, B, SQ, DM), jnp.float32),
            pltpu.SemaphoreType.DMA((N_DEV - 1,)),
            pltpu.SemaphoreType.DMA((N_DEV - 1,)),
            pltpu.SemaphoreType.DMA((N_DEV - 1,)),
            pltpu.SemaphoreType.DMA((N_DEV - 1,)),
        ],
        compiler_params=pltpu.CompilerParams(collective_id=0),
    )(xb, wqb, kb, vb, wob)


# baseline (device time: 103549 ns/iter reference)
import jax
import jax.numpy as jnp
from jax import lax
from jax.experimental import pallas as pl
from jax.experimental.pallas import tpu as pltpu

N_DEV = 4
B, SQ, DM = 2, 128, 512
HL, DH = 4, 64
SKV_SH = 128
WIN = 128
BF16 = jnp.bfloat16


def kernel(x, Wq, K_ext, V_ext, Wo):
    xb = x.astype(BF16)
    wqb = Wq.astype(BF16)
    wob = Wo.astype(BF16)
    kb = jnp.transpose(K_ext.astype(BF16), (2, 0, 1, 3))
    vb = jnp.transpose(V_ext.astype(BF16), (2, 0, 1, 3))

    def body(x_ref, wq_ref, k_ref, v_ref, wo_ref, out_ref,
             kv_comm, out_comm, kv_send, kv_recv, o_send, o_recv):
        my = lax.axis_index("i")
        left = lax.rem(my + N_DEV - 1, N_DEV)
        right = lax.rem(my + 1, N_DEV)

        barrier = pltpu.get_barrier_semaphore()
        for nbr in (left, right):
            pl.semaphore_signal(barrier, inc=1, device_id=(nbr,),
                                device_id_type=pl.DeviceIdType.MESH)
        pl.semaphore_wait(barrier, 2)

        kv_comm[0, 0] = k_ref[...]
        kv_comm[0, 1] = v_ref[...]

        qh = None
        for h in range(N_DEV - 1):
            rdma = pltpu.make_async_remote_copy(
                src_ref=kv_comm.at[h],
                dst_ref=kv_comm.at[h + 1],
                send_sem=kv_send.at[h],
                recv_sem=kv_recv.at[h],
                device_id=(right,),
                device_id_type=pl.DeviceIdType.MESH,
            )
            rdma.start()
            if h == 0:
                q = jnp.dot(x_ref[...].reshape(B * SQ, DM), wq_ref[...],
                            preferred_element_type=jnp.float32)
                qh = q.astype(BF16).reshape(B, SQ, HL, DH).transpose(2, 0, 1, 3)
            rdma.wait()

        h0 = HL * my
        s0 = my
        s1 = left
        k0 = kv_comm[pl.ds(s0, 1), 0, pl.ds(h0, HL)].reshape(HL, B, SKV_SH, DH)
        k1 = kv_comm[pl.ds(s1, 1), 0, pl.ds(h0, HL)].reshape(HL, B, SKV_SH, DH)
        v0 = kv_comm[pl.ds(s0, 1), 1, pl.ds(h0, HL)].reshape(HL, B, SKV_SH, DH)
        v1 = kv_comm[pl.ds(s1, 1), 1, pl.ds(h0, HL)].reshape(HL, B, SKV_SH, DH)
        k01 = jnp.concatenate([k0, k1], axis=2).reshape(HL * B, 2 * SKV_SH, DH)
        v01 = jnp.concatenate([v0, v1], axis=2).reshape(HL * B, 2 * SKV_SH, DH)
        qg = qh.reshape(HL * B, SQ, DH)

        scores = jnp.einsum('gsd,gtd->gst', qg, k01,
                            preferred_element_type=jnp.float32) * 0.125
        si = lax.broadcasted_iota(jnp.int32, (SQ, 2 * SKV_SH), 0)
        ti = lax.broadcasted_iota(jnp.int32, (SQ, 2 * SKV_SH), 1)
        mask = (ti - si) <= WIN
        scores = jnp.where(mask[None], scores, -1e9)

        m = jnp.max(scores, axis=-1, keepdims=True)
        e = jnp.exp(scores - m)
        w = (e / jnp.sum(e, axis=-1, keepdims=True)).astype(BF16)

        ctx = jnp.einsum('gst,gtd->gsd', w, v01,
                         preferred_element_type=jnp.float32)
        ctx2 = (ctx.astype(BF16).reshape(HL, B, SQ, DH)
                .transpose(1, 2, 0, 3).reshape(B * SQ, HL * DH))
        part = jnp.dot(ctx2, wo_ref[...], preferred_element_type=jnp.float32)
        out_comm[0] = part.reshape(B, SQ, DM)

        for h in range(N_DEV - 1):
            rdma = pltpu.make_async_remote_copy(
                src_ref=out_comm.at[h],
                dst_ref=out_comm.at[h + 1],
                send_sem=o_send.at[h],
                recv_sem=o_recv.at[h],
                device_id=(right,),
                device_id_type=pl.DeviceIdType.MESH,
            )
            rdma.start()
            rdma.wait()

        out_ref[...] = ((out_comm[0] + out_comm[1])
                        + (out_comm[2] + out_comm[3]))

    return pl.pallas_call(
        body,
        out_shape=jax.ShapeDtypeStruct((B, SQ, DM), jnp.float32),
        in_specs=[pl.BlockSpec(memory_space=pltpu.VMEM)] * 5,
        out_specs=pl.BlockSpec(memory_space=pltpu.VMEM),
        scratch_shapes=[
            pltpu.VMEM((N_DEV, 2, N_DEV * HL, B, SKV_SH, DH), BF16),
            pltpu.VMEM((N_DEV, B, SQ, DM), jnp.float32),
            pltpu.SemaphoreType.DMA((N_DEV - 1,)),
            pltpu.SemaphoreType.DMA((N_DEV - 1,)),
            pltpu.SemaphoreType.DMA((N_DEV - 1,)),
            pltpu.SemaphoreType.DMA((N_DEV - 1,)),
        ],
        compiler_params=pltpu.CompilerParams(collective_id=0),
    )(xb, wqb, kb, vb, wob)


# device time: 37292 ns/iter; 2.7767x vs baseline; 2.7767x over previous
import jax
import jax.numpy as jnp
from jax import lax
from jax.experimental import pallas as pl
from jax.experimental.pallas import tpu as pltpu

N_DEV = 4
B, SQ, DM = 2, 128, 512
HL, DH = 4, 64
SKV_SH = 128
WIN = 128
BF16 = jnp.bfloat16


def kernel(x, Wq, K_ext, V_ext, Wo):
    xb = x.astype(BF16)
    wqb = Wq.astype(BF16)
    wob = Wo.astype(BF16)
    kb = jnp.transpose(K_ext.astype(BF16), (2, 0, 1, 3))
    vb = jnp.transpose(V_ext.astype(BF16), (2, 0, 1, 3))

    def body(x_ref, wq_ref, k_ref, v_ref, wo_ref, out_ref,
             kv_mine, fwd_buf, out_parts,
             kvr, fwdr, kvs, o_send, o_recv):
        my = lax.axis_index("i")
        left = lax.rem(my + N_DEV - 1, N_DEV)
        right = lax.rem(my + 1, N_DEV)

        def copy(src, dst, ssem, rsem, tgt):
            return pltpu.make_async_remote_copy(
                src_ref=src, dst_ref=dst, send_sem=ssem, recv_sem=rsem,
                device_id=(tgt,), device_id_type=pl.DeviceIdType.MESH)

        def hgrp(ref, g):
            return ref.at[pl.ds(HL * g, HL)]

        barrier = pltpu.get_barrier_semaphore()
        for nbr in (left, right):
            pl.semaphore_signal(barrier, inc=1, device_id=(nbr,),
                                device_id_type=pl.DeviceIdType.MESH)
        pl.semaphore_wait(barrier, 2)

        @pl.when(my == 0)
        def _():
            kv_mine[0, 0] = k_ref[pl.ds(0, HL)]
            kv_mine[0, 1] = v_ref[pl.ds(0, HL)]
            copy(hgrp(k_ref, 2), fwd_buf.at[0], kvs.at[0], fwdr.at[0], 1).start()
            copy(hgrp(v_ref, 2), fwd_buf.at[1], kvs.at[1], fwdr.at[1], 1).start()
            copy(hgrp(k_ref, 1), kv_mine.at[0, 0], kvs.at[2], kvr.at[0, 0], 1).start()
            copy(hgrp(v_ref, 1), kv_mine.at[0, 1], kvs.at[3], kvr.at[0, 1], 1).start()
            copy(hgrp(k_ref, 3), kv_mine.at[0, 0], kvs.at[4], kvr.at[0, 0], 3).start()
            copy(hgrp(v_ref, 3), kv_mine.at[0, 1], kvs.at[5], kvr.at[0, 1], 3).start()

        @pl.when(my == 1)
        def _():
            kv_mine[1, 0] = k_ref[pl.ds(HL, HL)]
            kv_mine[1, 1] = v_ref[pl.ds(HL, HL)]
            copy(hgrp(k_ref, 3), fwd_buf.at[0], kvs.at[0], fwdr.at[0], 2).start()
            copy(hgrp(v_ref, 3), fwd_buf.at[1], kvs.at[1], fwdr.at[1], 2).start()
            copy(hgrp(k_ref, 2), kv_mine.at[1, 0], kvs.at[2], kvr.at[1, 0], 2).start()
            copy(hgrp(v_ref, 2), kv_mine.at[1, 1], kvs.at[3], kvr.at[1, 1], 2).start()
            copy(hgrp(k_ref, 0), kv_mine.at[1, 0], kvs.at[4], kvr.at[1, 0], 0).start()
            copy(hgrp(v_ref, 0), kv_mine.at[1, 1], kvs.at[5], kvr.at[1, 1], 0).start()

        q = jnp.dot(x_ref[...].reshape(B * SQ, DM), wq_ref[...],
                    preferred_element_type=jnp.float32)
        qg = (q.astype(BF16).reshape(B, SQ, HL, DH)
              .transpose(2, 0, 1, 3).reshape(HL * B, SQ, DH))

        @pl.when(my == 1)
        def _():
            copy(fwd_buf.at[0], fwd_buf.at[0], kvs.at[6], fwdr.at[0], 0).wait_recv()
            copy(fwd_buf.at[1], fwd_buf.at[1], kvs.at[6], fwdr.at[1], 0).wait_recv()
            copy(fwd_buf.at[0], kv_mine.at[0, 0], kvs.at[6], kvr.at[0, 0], 2).start()
            copy(fwd_buf.at[1], kv_mine.at[0, 1], kvs.at[7], kvr.at[0, 1], 2).start()

        @pl.when(my == 2)
        def _():
            copy(fwd_buf.at[0], fwd_buf.at[0], kvs.at[6], fwdr.at[0], 1).wait_recv()
            copy(fwd_buf.at[1], fwd_buf.at[1], kvs.at[6], fwdr.at[1], 1).wait_recv()
            copy(fwd_buf.at[0], kv_mine.at[1, 0], kvs.at[0], kvr.at[1, 0], 3).start()
            copy(fwd_buf.at[1], kv_mine.at[1, 1], kvs.at[1], kvr.at[1, 1], 3).start()

        def wait_chunk(c):
            copy(kv_mine.at[c, 0], kv_mine.at[c, 0], kvs.at[7], kvr.at[c, 0], 0).wait_recv()
            copy(kv_mine.at[c, 1], kv_mine.at[c, 1], kvs.at[7], kvr.at[c, 1], 0).wait_recv()

        @pl.when(my == 0)
        def _():
            wait_chunk(1)

        @pl.when(my == 1)
        def _():
            wait_chunk(0)

        @pl.when(my >= 2)
        def _():
            wait_chunk(0)
            wait_chunk(1)

        k01 = jnp.concatenate([kv_mine[0, 0], kv_mine[1, 0]],
                              axis=2).reshape(HL * B, 2 * SKV_SH, DH)
        v01 = jnp.concatenate([kv_mine[0, 1], kv_mine[1, 1]],
                              axis=2).reshape(HL * B, 2 * SKV_SH, DH)

        scores = jnp.einsum('gsd,gtd->gst', qg, k01,
                            preferred_element_type=jnp.float32) * 0.125
        si = lax.broadcasted_iota(jnp.int32, (SQ, 2 * SKV_SH), 0)
        ti = lax.broadcasted_iota(jnp.int32, (SQ, 2 * SKV_SH), 1)
        mask = (ti - si) <= WIN
        scores = jnp.where(mask[None], scores, -1e9)

        m = jnp.max(scores, axis=-1, keepdims=True)
        e = jnp.exp(scores - m)
        w = (e / jnp.sum(e, axis=-1, keepdims=True)).astype(BF16)

        ctx = jnp.einsum('gst,gtd->gsd', w, v01,
                         preferred_element_type=jnp.float32)
        ctx2 = (ctx.astype(BF16).reshape(HL, B, SQ, DH)
                .transpose(1, 2, 0, 3).reshape(B * SQ, HL * DH))
        part = jnp.dot(ctx2, wo_ref[...], preferred_element_type=jnp.float32)
        out_parts[0] = part.astype(BF16).reshape(B, SQ, DM)

        s_r = copy(out_parts.at[0], out_parts.at[1], o_send.at[0],
                   o_recv.at[0], right)
        s_l = copy(out_parts.at[0], out_parts.at[2], o_send.at[1],
                   o_recv.at[1], left)
        s_r.start()
        s_l.start()
        s_r.wait_recv()
        fwd = copy(out_parts.at[1], out_parts.at[3], o_send.at[2],
                   o_recv.at[2], right)
        fwd.start()
        s_l.wait_recv()
        fwd.wait_recv()

        out_ref[...] = ((out_parts[0].astype(jnp.float32)
                         + out_parts[1].astype(jnp.float32))
                        + (out_parts[2].astype(jnp.float32)
                           + out_parts[3].astype(jnp.float32)))

        s_r.wait_send()
        s_l.wait_send()
        fwd.wait_send()

        def wait_sends(n):
            for i in range(n):
                copy(fwd_buf.at[0], fwd_buf.at[0], kvs.at[i],
                     fwdr.at[0], 0).wait_send()

        @pl.when(my == 0)
        def _():
            wait_sends(6)

        @pl.when(my == 1)
        def _():
            wait_sends(8)

        @pl.when(my == 2)
        def _():
            wait_sends(2)

    return pl.pallas_call(
        body,
        out_shape=jax.ShapeDtypeStruct((B, SQ, DM), jnp.float32),
        in_specs=[pl.BlockSpec(memory_space=pltpu.VMEM)] * 5,
        out_specs=pl.BlockSpec(memory_space=pltpu.VMEM),
        scratch_shapes=[
            pltpu.VMEM((2, 2, HL, B, SKV_SH, DH), BF16),
            pltpu.VMEM((2, HL, B, SKV_SH, DH), BF16),
            pltpu.VMEM((N_DEV, B, SQ, DM), BF16),
            pltpu.SemaphoreType.DMA((2, 2)),
            pltpu.SemaphoreType.DMA((2,)),
            pltpu.SemaphoreType.DMA((8,)),
            pltpu.SemaphoreType.DMA((3,)),
            pltpu.SemaphoreType.DMA((3,)),
        ],
        compiler_params=pltpu.CompilerParams(collective_id=0),
    )(xb, wqb, kb, vb, wob)


# device time: 6761 ns/iter; 15.3156x vs baseline; 5.5158x over previous
import jax
import jax.numpy as jnp
from jax import lax
from jax.experimental import pallas as pl
from jax.experimental.pallas import tpu as pltpu

N_DEV = 4
B, SQ, DM = 2, 128, 512
HL, DH = 4, 64
SKV_SH = 128
WIN = 128
BF16 = jnp.bfloat16


def kernel(x, Wq, K_ext, V_ext, Wo):
    xb = x.astype(BF16)
    wqb = Wq.astype(BF16)
    wob = Wo.astype(BF16)
    kb = jnp.transpose(K_ext.astype(BF16), (2, 0, 1, 3))
    vb = jnp.transpose(V_ext.astype(BF16), (2, 0, 1, 3))

    def body(x_ref, wq_ref, k_ref, v_ref, wo_ref, out_ref,
             kv_mine, out_parts, kvr, kvs, o_send, o_recv):
        my = lax.axis_index("i")
        left = lax.rem(my + N_DEV - 1, N_DEV)
        right = lax.rem(my + 1, N_DEV)
        diag = lax.rem(my + 2, N_DEV)

        def copy(src, dst, ssem, rsem, tgt):
            return pltpu.make_async_remote_copy(
                src_ref=src, dst_ref=dst, send_sem=ssem, recv_sem=rsem,
                device_id=(tgt,), device_id_type=pl.DeviceIdType.MESH)

        def hgrp(ref, g):
            return ref.at[pl.ds(HL * g, HL)]

        barrier = pltpu.get_barrier_semaphore()
        for nbr in (left, right, diag):
            pl.semaphore_signal(barrier, inc=1, device_id=(nbr,),
                                device_id_type=pl.DeviceIdType.MESH)
        pl.semaphore_wait(barrier, 3)

        def kv_sends(me, chunk):
            for i, t in enumerate(((me + 2) % N_DEV, (me + 3) % N_DEV,
                                   (me + 1) % N_DEV)):
                copy(hgrp(k_ref, t), kv_mine.at[chunk, 0],
                     kvs.at[2 * i], kvr.at[chunk, 0], t).start()
                copy(hgrp(v_ref, t), kv_mine.at[chunk, 1],
                     kvs.at[2 * i + 1], kvr.at[chunk, 1], t).start()

        @pl.when(my == 0)
        def _():
            kv_mine[0, 0] = k_ref[pl.ds(0, HL)]
            kv_mine[0, 1] = v_ref[pl.ds(0, HL)]
            kv_sends(0, 0)

        @pl.when(my == 1)
        def _():
            kv_mine[1, 0] = k_ref[pl.ds(HL, HL)]
            kv_mine[1, 1] = v_ref[pl.ds(HL, HL)]
            kv_sends(1, 1)

        q = jnp.dot(x_ref[...].reshape(B * SQ, DM), wq_ref[...],
                    preferred_element_type=jnp.float32)
        qg = (q.astype(BF16).reshape(B, SQ, HL, DH)
              .transpose(2, 0, 1, 3).reshape(HL * B, SQ, DH))

        def wait_chunk(c):
            copy(kv_mine.at[c, 0], kv_mine.at[c, 0], kvs.at[7], kvr.at[c, 0], 0).wait_recv()
            copy(kv_mine.at[c, 1], kv_mine.at[c, 1], kvs.at[7], kvr.at[c, 1], 0).wait_recv()

        @pl.when(my == 0)
        def _():
            wait_chunk(1)

        @pl.when(my == 1)
        def _():
            wait_chunk(0)

        @pl.when(my >= 2)
        def _():
            wait_chunk(0)
            wait_chunk(1)

        k01 = jnp.concatenate([kv_mine[0, 0], kv_mine[1, 0]],
                              axis=2).reshape(HL * B, 2 * SKV_SH, DH)
        v01 = jnp.concatenate([kv_mine[0, 1], kv_mine[1, 1]],
                              axis=2).reshape(HL * B, 2 * SKV_SH, DH)

        scores = jnp.einsum('gsd,gtd->gst', qg, k01,
                            preferred_element_type=jnp.float32) * 0.125
        si = lax.broadcasted_iota(jnp.int32, (SQ, 2 * SKV_SH), 0)
        ti = lax.broadcasted_iota(jnp.int32, (SQ, 2 * SKV_SH), 1)
        mask = (ti - si) <= WIN
        scores = jnp.where(mask[None], scores, -1e9)

        m = jnp.max(scores, axis=-1, keepdims=True)
        e = jnp.exp(scores - m)
        w = (e / jnp.sum(e, axis=-1, keepdims=True)).astype(BF16)

        ctx = jnp.einsum('gst,gtd->gsd', w, v01,
                         preferred_element_type=jnp.float32)
        ctx2 = (ctx.astype(BF16).reshape(HL, B, SQ, DH)
                .transpose(1, 2, 0, 3).reshape(B * SQ, HL * DH))
        part = jnp.dot(ctx2, wo_ref[...], preferred_element_type=jnp.float32)
        out_parts[0] = part.astype(BF16).reshape(B, SQ, DM)

        s_r = copy(out_parts.at[0], out_parts.at[1], o_send.at[0],
                   o_recv.at[0], right)
        s_l = copy(out_parts.at[0], out_parts.at[2], o_send.at[1],
                   o_recv.at[1], left)
        s_d = copy(out_parts.at[0], out_parts.at[3], o_send.at[2],
                   o_recv.at[2], diag)
        s_d.start()
        s_r.start()
        s_l.start()
        s_r.wait_recv()
        s_l.wait_recv()
        s_d.wait_recv()

        out_ref[...] = ((out_parts[0].astype(jnp.float32)
                         + out_parts[1].astype(jnp.float32))
                        + (out_parts[2].astype(jnp.float32)
                           + out_parts[3].astype(jnp.float32)))

        s_r.wait_send()
        s_l.wait_send()
        s_d.wait_send()

        @pl.when(my <= 1)
        def _():
            for i in range(6):
                copy(kv_mine.at[0, 0], kv_mine.at[0, 0], kvs.at[i],
                     kvr.at[0, 0], 0).wait_send()

    return pl.pallas_call(
        body,
        out_shape=jax.ShapeDtypeStruct((B, SQ, DM), jnp.float32),
        in_specs=[pl.BlockSpec(memory_space=pltpu.VMEM)] * 5,
        out_specs=pl.BlockSpec(memory_space=pltpu.VMEM),
        scratch_shapes=[
            pltpu.VMEM((2, 2, HL, B, SKV_SH, DH), BF16),
            pltpu.VMEM((N_DEV, B, SQ, DM), BF16),
            pltpu.SemaphoreType.DMA((2, 2)),
            pltpu.SemaphoreType.DMA((8,)),
            pltpu.SemaphoreType.DMA((3,)),
            pltpu.SemaphoreType.DMA((3,)),
        ],
        compiler_params=pltpu.CompilerParams(collective_id=0),
    )(xb, wqb, kb, vb, wob)
